# baseline (speedup 1.0000x reference)
import functools

import jax
import jax.numpy as jnp
from jax import lax
from jax.experimental import pallas as pl
from jax.experimental.pallas import tpu as pltpu
from jax.experimental.pallas import tpu_sc as plsc

_VOCAB = 100000
_EMBED = 64
_BATCH = 1024


@functools.lru_cache(maxsize=None)
def _make_sc_gather(B: int, D: int):
    info = plsc.get_sparse_core_info()
    nc, ns = info.num_cores, info.num_subcores
    nw = nc * ns
    assert B % (8 * nw) == 0
    b_per_w = B // nw
    mesh = plsc.VectorSubcoreMesh(core_axis_name="c", subcore_axis_name="s")

    @functools.partial(
        pl.kernel,
        mesh=mesh,
        compiler_params=pltpu.CompilerParams(use_tc_tiling_on_sc=False),
        out_type=jax.ShapeDtypeStruct((B, D), jnp.float32),
        scratch_types=[
            pltpu.VMEM((b_per_w,), jnp.int32),
            pltpu.VMEM((b_per_w, D), jnp.float32),
            pltpu.SemaphoreType.DMA,
        ],
    )
    def gather(idx_hbm, table_hbm, out_hbm, idx_v, rows_v, sem):
        wid = lax.axis_index("s") * nc + lax.axis_index("c")
        base = wid * b_per_w
        pltpu.sync_copy(idx_hbm.at[pl.ds(base, b_per_w)], idx_v)
        pltpu.async_copy(table_hbm.at[idx_v], rows_v, sem).wait()
        pltpu.sync_copy(rows_v, out_hbm.at[pl.ds(base, b_per_w)])

    return gather


_VT = 2048


def _mm_body(e_ref, w_ref, b_ref, o_ref):
    acc = lax.dot_general(
        e_ref[...],
        w_ref[...],
        (((1,), (1,)), ((), ())),
        preferred_element_type=jnp.float32,
    )
    o_ref[...] = acc + b_ref[...]


def _projection(e, W, b2):
    B, D = e.shape
    V = W.shape[0]
    nt = pl.cdiv(V, _VT)
    return pl.pallas_call(
        _mm_body,
        grid=(nt,),
        in_specs=[
            pl.BlockSpec((B, D), lambda i: (0, 0)),
            pl.BlockSpec((_VT, D), lambda i: (i, 0)),
            pl.BlockSpec((1, _VT), lambda i: (0, i)),
        ],
        out_specs=pl.BlockSpec((B, _VT), lambda i: (0, i)),
        out_shape=jax.ShapeDtypeStruct((B, V), jnp.float32),
    )(e, W, b2)


def kernel(x, emb_table, W, b):
    idx = x.astype(jnp.int32)
    e = _make_sc_gather(_BATCH, _EMBED)(idx, emb_table)
    return _projection(e, W, b.reshape(1, _VOCAB))


# speedup vs baseline: 1.0024x; 1.0024x over previous
import functools

import jax
import jax.numpy as jnp
from jax import lax
from jax.experimental import pallas as pl
from jax.experimental.pallas import tpu as pltpu
from jax.experimental.pallas import tpu_sc as plsc

_VOCAB = 100000
_EMBED = 64
_BATCH = 1024


@functools.lru_cache(maxsize=None)
def _make_sc_gather(B: int, D: int):
    info = plsc.get_sparse_core_info()
    nc, ns = info.num_cores, info.num_subcores
    nw = nc * ns
    assert B % (8 * nw) == 0
    b_per_w = B // nw
    mesh = plsc.VectorSubcoreMesh(core_axis_name="c", subcore_axis_name="s")

    @functools.partial(
        pl.kernel,
        mesh=mesh,
        compiler_params=pltpu.CompilerParams(use_tc_tiling_on_sc=False),
        out_type=jax.ShapeDtypeStruct((B, D), jnp.float32),
        scratch_types=[
            pltpu.VMEM((b_per_w,), jnp.int32),
            pltpu.VMEM((b_per_w, D), jnp.float32),
            pltpu.SemaphoreType.DMA,
        ],
    )
    def gather(idx_hbm, table_hbm, out_hbm, idx_v, rows_v, sem):
        wid = lax.axis_index("s") * nc + lax.axis_index("c")
        base = wid * b_per_w
        pltpu.sync_copy(idx_hbm.at[pl.ds(base, b_per_w)], idx_v)
        pltpu.async_copy(table_hbm.at[idx_v], rows_v, sem).wait()
        pltpu.sync_copy(rows_v, out_hbm.at[pl.ds(base, b_per_w)])

    return gather


_VT = 4096


def _mm_body(e_ref, w_ref, b_ref, o_ref):
    acc = lax.dot_general(
        e_ref[...],
        w_ref[...],
        (((1,), (1,)), ((), ())),
        preferred_element_type=jnp.float32,
    )
    o_ref[...] = acc + b_ref[...]


def _projection(e, W, b2):
    B, D = e.shape
    V = W.shape[0]
    nt = pl.cdiv(V, _VT)
    return pl.pallas_call(
        _mm_body,
        grid=(nt,),
        in_specs=[
            pl.BlockSpec((B, D), lambda i: (0, 0)),
            pl.BlockSpec((_VT, D), lambda i: (i, 0)),
            pl.BlockSpec((1, _VT), lambda i: (0, i)),
        ],
        out_specs=pl.BlockSpec((B, _VT), lambda i: (0, i)),
        out_shape=jax.ShapeDtypeStruct((B, V), jnp.float32),
    )(e, W, b2)


def kernel(x, emb_table, W, b):
    idx = x.astype(jnp.int32)
    e = _make_sc_gather(_BATCH, _EMBED)(idx, emb_table)
    return _projection(e, W, b.reshape(1, _VOCAB))


# speedup vs baseline: 1.0094x; 1.0070x over previous
import functools

import jax
import jax.numpy as jnp
from jax import lax
from jax.experimental import pallas as pl
from jax.experimental.pallas import tpu as pltpu
from jax.experimental.pallas import tpu_sc as plsc

_VOCAB = 100000
_EMBED = 64
_BATCH = 1024

_VT = 2048
_NFULL = _VOCAB // _VT
_TAIL = _VOCAB - _NFULL * _VT
_P = 4
_RB = _BATCH // _P


@functools.lru_cache(maxsize=None)
def _make_sc_gather(B: int, D2: int):
    info = plsc.get_sparse_core_info()
    nc, ns = info.num_cores, info.num_subcores
    nw = nc * ns
    assert B % (8 * nw) == 0
    b_per_w = B // nw
    mesh = plsc.VectorSubcoreMesh(core_axis_name="c", subcore_axis_name="s")

    @functools.partial(
        pl.kernel,
        mesh=mesh,
        out_type=jax.ShapeDtypeStruct((B, D2), jnp.float32),
        scratch_types=[
            pltpu.VMEM((b_per_w,), jnp.int32),
            pltpu.VMEM((b_per_w, D2), jnp.float32),
            pltpu.SemaphoreType.DMA,
        ],
    )
    def gather(idx_hbm, table_hbm, out_hbm, idx_v, rows_v, sem):
        wid = lax.axis_index("s") * nc + lax.axis_index("c")
        base = wid * b_per_w
        pltpu.sync_copy(idx_hbm.at[pl.ds(base, b_per_w)], idx_v)
        pltpu.async_copy(table_hbm.at[idx_v], rows_v, sem).wait()
        pltpu.sync_copy(rows_v, out_hbm.at[pl.ds(base, b_per_w)])

    return gather


def _mm_body(e_ref, par_ref, w_ref, b_ref, out_hbm, obuf, tbuf, sem, tsem):
    i = pl.program_id(0)
    e = jnp.where(par_ref[...] != 0, e_ref[:, _EMBED:], e_ref[:, :_EMBED])
    acc = lax.dot_general(
        e, w_ref[...], (((1,), (1,)), ((), ())),
        preferred_element_type=jnp.float32,
    ) + b_ref[...]
    slot = lax.rem(i, 2)

    @pl.when(i < _NFULL)
    def _main():
        @pl.when(i >= 2)
        def _wait_prev():
            for p in range(_P):
                pltpu.make_async_copy(
                    obuf.at[slot, pl.ds(p * _RB, _RB), :],
                    out_hbm.at[pl.ds(p * _RB, _RB), pl.ds((i - 2) * _VT, _VT)],
                    sem.at[slot, p],
                ).wait()

        @pl.when(slot == 0)
        def _st0():
            obuf[0] = acc

        @pl.when(slot == 1)
        def _st1():
            obuf[1] = acc

        for p in range(_P):
            pltpu.make_async_copy(
                obuf.at[slot, pl.ds(p * _RB, _RB), :],
                out_hbm.at[pl.ds(p * _RB, _RB), pl.ds(i * _VT, _VT)],
                sem.at[slot, p],
            ).start()

    @pl.when(i == _NFULL)
    def _tail():
        for s in range(2):
            for p in range(_P):
                pltpu.make_async_copy(
                    obuf.at[s, pl.ds(p * _RB, _RB), :],
                    out_hbm.at[pl.ds(p * _RB, _RB),
                               pl.ds((_NFULL - 2 + s) * _VT, _VT)],
                    sem.at[s, p],
                ).wait()
        tbuf[...] = acc[:, :_TAIL]
        for p in range(_P):
            pltpu.make_async_copy(
                tbuf.at[pl.ds(p * _RB, _RB), :],
                out_hbm.at[pl.ds(p * _RB, _RB), pl.ds(_NFULL * _VT, _TAIL)],
                tsem.at[p],
            ).start()
        for p in range(_P):
            pltpu.make_async_copy(
                tbuf.at[pl.ds(p * _RB, _RB), :],
                out_hbm.at[pl.ds(p * _RB, _RB), pl.ds(_NFULL * _VT, _TAIL)],
                tsem.at[p],
            ).wait()


def _projection(e2, par, W, b2):
    B = e2.shape[0]
    V = W.shape[0]
    return pl.pallas_call(
        _mm_body,
        grid=(_NFULL + 1,),
        in_specs=[
            pl.BlockSpec((B, 2 * _EMBED), lambda i: (0, 0)),
            pl.BlockSpec((B, 1), lambda i: (0, 0)),
            pl.BlockSpec((_VT, _EMBED), lambda i: (i, 0)),
            pl.BlockSpec((1, _VT), lambda i: (0, i)),
        ],
        out_specs=pl.BlockSpec(memory_space=pl.ANY),
        out_shape=jax.ShapeDtypeStruct((B, V), jnp.float32),
        scratch_shapes=[
            pltpu.VMEM((2, B, _VT), jnp.float32),
            pltpu.VMEM((B, _TAIL), jnp.float32),
            pltpu.SemaphoreType.DMA((2, _P)),
            pltpu.SemaphoreType.DMA((_P,)),
        ],
        compiler_params=pltpu.CompilerParams(
            dimension_semantics=("arbitrary",),
        ),
    )(e2, par, W, b2)


def kernel(x, emb_table, W, b):
    idx = x.astype(jnp.int32)
    table2 = emb_table.reshape(_VOCAB // 2, 2 * _EMBED)
    e2 = _make_sc_gather(_BATCH, 2 * _EMBED)(idx >> 1, table2)
    par = (idx & 1).reshape(_BATCH, 1)
    return _projection(e2, par, W, b.reshape(1, _VOCAB))


# speedup vs baseline: 2.2190x; 2.1983x over previous
import functools

import jax
import jax.numpy as jnp
from jax import lax
from jax.experimental import pallas as pl
from jax.experimental.pallas import tpu as pltpu
from jax.experimental.pallas import tpu_sc as plsc

_VOCAB = 100000
_EMBED = 64
_BATCH = 1024

_VT = 2048


@functools.lru_cache(maxsize=None)
def _make_sc_gather(B: int, D: int):
    info = plsc.get_sparse_core_info()
    nc, ns = info.num_cores, info.num_subcores
    nw = nc * ns
    assert B % (8 * nw) == 0
    b_per_w = B // nw
    mesh = plsc.VectorSubcoreMesh(core_axis_name="c", subcore_axis_name="s")

    @functools.partial(
        pl.kernel,
        mesh=mesh,
        compiler_params=pltpu.CompilerParams(use_tc_tiling_on_sc=False),
        out_type=jax.ShapeDtypeStruct((B, D), jnp.float32),
        scratch_types=[
            pltpu.VMEM((b_per_w,), jnp.int32),
            pltpu.VMEM((b_per_w, D), jnp.float32),
            pltpu.SemaphoreType.DMA,
        ],
    )
    def gather(idx_hbm, table_hbm, out_hbm, idx_v, rows_v, sem):
        wid = lax.axis_index("s") * nc + lax.axis_index("c")
        base = wid * b_per_w
        pltpu.sync_copy(idx_hbm.at[pl.ds(base, b_per_w)], idx_v)
        pltpu.async_copy(table_hbm.at[idx_v], rows_v, sem).wait()
        pltpu.sync_copy(rows_v, out_hbm.at[pl.ds(base, b_per_w)])

    return gather


def _mm_body(wt_ref, e_ref, b_ref, o_ref):
    acc = lax.dot_general(
        wt_ref[...],
        e_ref[...],
        (((0,), (1,)), ((), ())),
        preferred_element_type=jnp.float32,
    )
    o_ref[...] = acc + b_ref[...]


def _projection_t(wt, e, b2):
    D, V = wt.shape
    B = e.shape[0]
    nt = pl.cdiv(V, _VT)
    return pl.pallas_call(
        _mm_body,
        grid=(nt,),
        in_specs=[
            pl.BlockSpec((D, _VT), lambda i: (0, i)),
            pl.BlockSpec((B, D), lambda i: (0, 0)),
            pl.BlockSpec((_VT, 1), lambda i: (i, 0)),
        ],
        out_specs=pl.BlockSpec((_VT, B), lambda i: (i, 0)),
        out_shape=jax.ShapeDtypeStruct((V, B), jnp.float32),
        compiler_params=pltpu.CompilerParams(
            dimension_semantics=("arbitrary",),
        ),
    )(wt, e, b2)


def kernel(x, emb_table, W, b):
    idx = x.astype(jnp.int32)
    e = _make_sc_gather(_BATCH, _EMBED)(idx, emb_table)
    out_t = _projection_t(W.T, e, b.reshape(_VOCAB, 1))
    return out_t.T


# speedup vs baseline: 3.1335x; 1.4122x over previous
import functools

import jax
import jax.numpy as jnp
from jax import lax
from jax.experimental import pallas as pl
from jax.experimental.pallas import tpu as pltpu
from jax.experimental.pallas import tpu_sc as plsc

_VOCAB = 100000
_EMBED = 64
_BATCH = 1024

_VT = 2048


_CHUNK = 128


@functools.lru_cache(maxsize=None)
def _make_sc_gather(B: int, D: int, V: int):
    info = plsc.get_sparse_core_info()
    nc, ns = info.num_cores, info.num_subcores
    nw = nc * ns
    assert B % (8 * nw) == 0
    b_per_w = B // nw
    n_per_w = b_per_w * D
    nchunk = n_per_w // _CHUNK
    mesh = plsc.VectorSubcoreMesh(core_axis_name="c", subcore_axis_name="s")

    @functools.partial(
        pl.kernel,
        mesh=mesh,
        compiler_params=pltpu.CompilerParams(use_tc_tiling_on_sc=False),
        out_type=jax.ShapeDtypeStruct((B * D,), jnp.float32),
        scratch_types=[
            pltpu.VMEM((n_per_w,), jnp.int32),
            pltpu.VMEM((n_per_w,), jnp.float32),
            pltpu.SemaphoreType.DMA,
        ],
    )
    def gather(idx_hbm, table_hbm, out_hbm, idx_v, rows_v, sem):
        wid = lax.axis_index("s") * nc + lax.axis_index("c")
        base = wid * n_per_w
        pltpu.sync_copy(idx_hbm.at[pl.ds(base, n_per_w)], idx_v)
        for c in range(nchunk):
            pltpu.async_copy(
                table_hbm.at[idx_v.at[pl.ds(c * _CHUNK, _CHUNK)]],
                rows_v.at[pl.ds(c * _CHUNK, _CHUNK)],
                sem,
            ).start()
        for c in range(nchunk):
            pltpu.make_async_copy(
                table_hbm.at[idx_v.at[pl.ds(c * _CHUNK, _CHUNK)]],
                rows_v.at[pl.ds(c * _CHUNK, _CHUNK)],
                sem,
            ).wait()
        pltpu.sync_copy(rows_v, out_hbm.at[pl.ds(base, n_per_w)])

    return gather


def _mm_body(wt_ref, e_ref, b_ref, o_ref):
    acc = lax.dot_general(
        wt_ref[...],
        e_ref[...],
        (((0,), (1,)), ((), ())),
        preferred_element_type=jnp.float32,
    )
    bias = lax.dot_general(
        b_ref[...],
        jnp.ones((e_ref.shape[0], 1), jnp.float32),
        (((0,), (1,)), ((), ())),
        preferred_element_type=jnp.float32,
    )
    o_ref[...] = acc + bias


def _projection_t(wt, e, brow):
    D, V = wt.shape
    B = e.shape[0]
    nt = pl.cdiv(V, _VT)
    return pl.pallas_call(
        _mm_body,
        grid=(nt,),
        in_specs=[
            pl.BlockSpec((D, _VT), lambda i: (0, i)),
            pl.BlockSpec((B, D), lambda i: (0, 0)),
            pl.BlockSpec((1, _VT), lambda i: (0, i)),
        ],
        out_specs=pl.BlockSpec((_VT, B), lambda i: (i, 0)),
        out_shape=jax.ShapeDtypeStruct((V, B), jnp.float32),
        compiler_params=pltpu.CompilerParams(
            dimension_semantics=("arbitrary",),
        ),
    )(wt, e, brow)


def kernel(x, emb_table, W, b):
    idx = x.astype(jnp.int32)
    table_flat = emb_table.T.reshape(_EMBED * _VOCAB)
    eidx = (jnp.arange(_EMBED, dtype=jnp.int32)[None, :] * _VOCAB
            + idx[:, None]).reshape(_BATCH * _EMBED)
    e_flat = _make_sc_gather(_BATCH, _EMBED, _VOCAB)(eidx, table_flat)
    e = e_flat.reshape(_BATCH, _EMBED)
    out_t = _projection_t(W.T, e, b.reshape(1, _VOCAB))
    return out_t.T


# speedup vs baseline: 3.5498x; 1.1328x over previous
import functools

import jax
import jax.numpy as jnp
from jax import lax
from jax.experimental import pallas as pl
from jax.experimental.pallas import tpu as pltpu
from jax.experimental.pallas import tpu_sc as plsc

_VOCAB = 100000
_EMBED = 64
_BATCH = 1024

_VT = 2048


_RND = 8


@functools.lru_cache(maxsize=None)
def _make_sc_gather(B: int, D: int, V: int):
    info = plsc.get_sparse_core_info()
    nc, ns, L = info.num_cores, info.num_subcores, info.num_lanes
    nw = nc * ns
    assert B % (8 * nw) == 0 and D % L == 0
    bpw = B // nw
    nrounds = bpw // _RND
    mesh = plsc.VectorSubcoreMesh(core_axis_name="c", subcore_axis_name="s")

    @functools.partial(
        pl.kernel,
        mesh=mesh,
        compiler_params=pltpu.CompilerParams(needs_layout_passes=False),
        out_type=jax.ShapeDtypeStruct((B, D), jnp.float32),
        scratch_types=[
            pltpu.VMEM((bpw,), jnp.int32),
            pltpu.VMEM((D, _RND * 128), jnp.float32),
            pltpu.VMEM((bpw, D), jnp.float32),
            pltpu.SemaphoreType.DMA,
        ],
    )
    def gather(idx_hbm, et_hbm, out_hbm, idx_v, tbuf, rows_v, sem):
        wid = lax.axis_index("s") * nc + lax.axis_index("c")
        base = wid * bpw
        pltpu.sync_copy(idx_hbm.at[pl.ds(base, bpw)], idx_v)
        dlanes = [lax.iota(jnp.int32, L) + k * L for k in range(D // L)]
        for r in range(nrounds):
            vvecs = [idx_v[pl.ds(((r * _RND + s) // L) * L, L)]
                     for s in range(_RND)]
            starts = []
            for s in range(_RND):
                v = vvecs[s][(r * _RND + s) % L]
                c128 = pl.multiple_of((v >> 7) * 128, 128)
                cp = pltpu.make_async_copy(
                    et_hbm.at[:, pl.ds(c128, 128)],
                    tbuf.at[:, pl.ds(s * 128, 128)],
                    sem,
                )
                cp.start()
                starts.append(cp)
            for cp in starts:
                cp.wait()
            for s in range(_RND):
                j = r * _RND + s
                v = vvecs[s][j % L]
                lane = jnp.full((L,), v & 127, jnp.int32) + s * 128
                for k in range(D // L):
                    vals = plsc.load_gather(tbuf, [dlanes[k], lane])
                    plsc.store_scatter(
                        rows_v, [jnp.full((L,), j, jnp.int32), dlanes[k]], vals
                    )
        pltpu.sync_copy(rows_v, out_hbm.at[pl.ds(base, bpw)])

    return gather


def _mm_body(wt_ref, e_ref, b_ref, o_ref):
    acc = lax.dot_general(
        wt_ref[...],
        e_ref[...],
        (((0,), (1,)), ((), ())),
        preferred_element_type=jnp.float32,
    )
    bias = lax.dot_general(
        b_ref[...],
        jnp.ones((e_ref.shape[0], 1), jnp.float32),
        (((0,), (1,)), ((), ())),
        preferred_element_type=jnp.float32,
    )
    o_ref[...] = acc + bias


def _projection_t(wt, e, brow):
    D, V = wt.shape
    B = e.shape[0]
    nt = pl.cdiv(V, _VT)
    return pl.pallas_call(
        _mm_body,
        grid=(nt,),
        in_specs=[
            pl.BlockSpec((D, _VT), lambda i: (0, i)),
            pl.BlockSpec((B, D), lambda i: (0, 0)),
            pl.BlockSpec((1, _VT), lambda i: (0, i)),
        ],
        out_specs=pl.BlockSpec((_VT, B), lambda i: (i, 0)),
        out_shape=jax.ShapeDtypeStruct((V, B), jnp.float32),
        compiler_params=pltpu.CompilerParams(
            dimension_semantics=("arbitrary",),
        ),
    )(wt, e, brow)


def kernel(x, emb_table, W, b):
    idx = x.astype(jnp.int32)
    e = _make_sc_gather(_BATCH, _EMBED, _VOCAB)(idx, emb_table.T)
    out_t = _projection_t(W.T, e, b.reshape(1, _VOCAB))
    return out_t.T


# speedup vs baseline: 3.5954x; 1.0129x over previous
import functools

import jax
import jax.numpy as jnp
from jax import lax
from jax.experimental import pallas as pl
from jax.experimental.pallas import tpu as pltpu
from jax.experimental.pallas import tpu_sc as plsc

_VOCAB = 100000
_EMBED = 64
_BATCH = 1024

_VT = 4096


_RND = 8


@functools.lru_cache(maxsize=None)
def _make_sc_gather(B: int, D: int, V: int):
    info = plsc.get_sparse_core_info()
    nc, ns, L = info.num_cores, info.num_subcores, info.num_lanes
    nw = nc * ns
    assert B % (8 * nw) == 0 and D % L == 0
    bpw = B // nw
    nrounds = bpw // _RND
    mesh = plsc.VectorSubcoreMesh(core_axis_name="c", subcore_axis_name="s")

    @functools.partial(
        pl.kernel,
        mesh=mesh,
        compiler_params=pltpu.CompilerParams(needs_layout_passes=False),
        out_type=jax.ShapeDtypeStruct((B, D), jnp.float32),
        scratch_types=[
            pltpu.VMEM((bpw,), jnp.int32),
            pltpu.VMEM((D, _RND * 128), jnp.float32),
            pltpu.VMEM((bpw, D), jnp.float32),
            pltpu.SemaphoreType.DMA,
        ],
    )
    def gather(idx_hbm, et_hbm, out_hbm, idx_v, tbuf, rows_v, sem):
        wid = lax.axis_index("s") * nc + lax.axis_index("c")
        base = wid * bpw
        pltpu.sync_copy(idx_hbm.at[pl.ds(base, bpw)], idx_v)
        dlanes = [lax.iota(jnp.int32, L) + k * L for k in range(D // L)]
        for r in range(nrounds):
            vvecs = [idx_v[pl.ds(((r * _RND + s) // L) * L, L)]
                     for s in range(_RND)]
            starts = []
            for s in range(_RND):
                v = vvecs[s][(r * _RND + s) % L]
                c128 = pl.multiple_of((v >> 7) * 128, 128)
                cp = pltpu.make_async_copy(
                    et_hbm.at[:, pl.ds(c128, 128)],
                    tbuf.at[:, pl.ds(s * 128, 128)],
                    sem,
                )
                cp.start()
                starts.append(cp)
            for cp in starts:
                cp.wait()
            for s in range(_RND):
                j = r * _RND + s
                v = vvecs[s][j % L]
                lane = jnp.full((L,), v & 127, jnp.int32) + s * 128
                for k in range(D // L):
                    vals = plsc.load_gather(tbuf, [dlanes[k], lane])
                    plsc.store_scatter(
                        rows_v, [jnp.full((L,), j, jnp.int32), dlanes[k]], vals
                    )
        pltpu.sync_copy(rows_v, out_hbm.at[pl.ds(base, bpw)])

    return gather


def _mm_body(wt_ref, e_ref, b_ref, o_ref):
    acc = lax.dot_general(
        wt_ref[...],
        e_ref[...],
        (((0,), (1,)), ((), ())),
        preferred_element_type=jnp.float32,
    )
    bias = lax.dot_general(
        b_ref[...],
        jnp.ones((e_ref.shape[0], 1), jnp.float32),
        (((0,), (1,)), ((), ())),
        preferred_element_type=jnp.float32,
    )
    o_ref[...] = acc + bias


def _projection_t(wt, e, brow):
    D, V = wt.shape
    B = e.shape[0]
    nt = pl.cdiv(V, _VT)
    return pl.pallas_call(
        _mm_body,
        grid=(nt,),
        in_specs=[
            pl.BlockSpec((D, _VT), lambda i: (0, i)),
            pl.BlockSpec((B, D), lambda i: (0, 0)),
            pl.BlockSpec((1, _VT), lambda i: (0, i)),
        ],
        out_specs=pl.BlockSpec((_VT, B), lambda i: (i, 0)),
        out_shape=jax.ShapeDtypeStruct((V, B), jnp.float32),
        compiler_params=pltpu.CompilerParams(
            dimension_semantics=("arbitrary",),
        ),
    )(wt, e, brow)


def kernel(x, emb_table, W, b):
    idx = x.astype(jnp.int32)
    e = _make_sc_gather(_BATCH, _EMBED, _VOCAB)(idx, emb_table.T)
    out_t = _projection_t(W.T, e, b.reshape(1, _VOCAB))
    return out_t.T


# speedup vs baseline: 3.6017x; 1.0017x over previous
import functools

import jax
import jax.numpy as jnp
from jax import lax
from jax.experimental import pallas as pl
from jax.experimental.pallas import tpu as pltpu
from jax.experimental.pallas import tpu_sc as plsc

_VOCAB = 100000
_EMBED = 64
_BATCH = 1024

_VT = 4096


_RND = 4


@functools.lru_cache(maxsize=None)
def _make_sc_gather(B: int, D: int, V: int):
    info = plsc.get_sparse_core_info()
    nc, ns, L = info.num_cores, info.num_subcores, info.num_lanes
    nw = nc * ns
    assert B % (8 * nw) == 0 and D % L == 0
    bpw = B // nw
    nrounds = bpw // _RND
    mesh = plsc.VectorSubcoreMesh(core_axis_name="c", subcore_axis_name="s")

    @functools.partial(
        pl.kernel,
        mesh=mesh,
        compiler_params=pltpu.CompilerParams(needs_layout_passes=False),
        out_type=jax.ShapeDtypeStruct((B, D), jnp.float32),
        scratch_types=[
            pltpu.VMEM((bpw,), jnp.int32),
            pltpu.VMEM((D, 2 * _RND * 128), jnp.float32),
            pltpu.VMEM((bpw, D), jnp.float32),
            pltpu.SemaphoreType.DMA((2,)),
        ],
    )
    def gather(idx_hbm, et_hbm, out_hbm, idx_v, tbuf, rows_v, sem):
        wid = lax.axis_index("s") * nc + lax.axis_index("c")
        base = wid * bpw
        pltpu.sync_copy(idx_hbm.at[pl.ds(base, bpw)], idx_v)
        dlanes = [lax.iota(jnp.int32, L) + k * L for k in range(D // L)]

        def vvec_of(r, s):
            return idx_v[pl.ds(((r * _RND + s) // L) * L, L)]

        def fire(r):
            half = r % 2
            for s in range(_RND):
                v = vvec_of(r, s)[(r * _RND + s) % L]
                c128 = pl.multiple_of((v >> 7) * 128, 128)
                pltpu.make_async_copy(
                    et_hbm.at[:, pl.ds(c128, 128)],
                    tbuf.at[:, pl.ds((half * _RND + s) * 128, 128)],
                    sem.at[half],
                ).start()

        def drain(r):
            half = r % 2
            for s in range(_RND):
                pltpu.make_async_copy(
                    et_hbm.at[:, pl.ds(0, 128)],
                    tbuf.at[:, pl.ds((half * _RND + s) * 128, 128)],
                    sem.at[half],
                ).wait()

        fire(0)
        for r in range(nrounds):
            if r + 1 < nrounds:
                fire(r + 1)
            drain(r)
            half = r % 2
            for s in range(_RND):
                j = r * _RND + s
                v = vvec_of(r, s)[j % L]
                lane = jnp.full((L,), v & 127, jnp.int32) + (half * _RND + s) * 128
                for k in range(D // L):
                    vals = plsc.load_gather(tbuf, [dlanes[k], lane])
                    plsc.store_scatter(
                        rows_v, [jnp.full((L,), j, jnp.int32), dlanes[k]], vals
                    )
        pltpu.sync_copy(rows_v, out_hbm.at[pl.ds(base, bpw)])

    return gather


def _mm_body(wt_ref, e_ref, b_ref, o_ref):
    acc = lax.dot_general(
        wt_ref[...],
        e_ref[...],
        (((0,), (1,)), ((), ())),
        preferred_element_type=jnp.float32,
    )
    bias = lax.dot_general(
        b_ref[...],
        jnp.ones((e_ref.shape[0], 1), jnp.float32),
        (((0,), (1,)), ((), ())),
        preferred_element_type=jnp.float32,
    )
    o_ref[...] = acc + bias


def _projection_t(wt, e, brow):
    D, V = wt.shape
    B = e.shape[0]
    nt = pl.cdiv(V, _VT)
    return pl.pallas_call(
        _mm_body,
        grid=(nt,),
        in_specs=[
            pl.BlockSpec((D, _VT), lambda i: (0, i)),
            pl.BlockSpec((B, D), lambda i: (0, 0)),
            pl.BlockSpec((1, _VT), lambda i: (0, i)),
        ],
        out_specs=pl.BlockSpec((_VT, B), lambda i: (i, 0)),
        out_shape=jax.ShapeDtypeStruct((V, B), jnp.float32),
        compiler_params=pltpu.CompilerParams(
            dimension_semantics=("arbitrary",),
        ),
    )(wt, e, brow)


def kernel(x, emb_table, W, b):
    idx = x.astype(jnp.int32)
    e = _make_sc_gather(_BATCH, _EMBED, _VOCAB)(idx, emb_table.T)
    out_t = _projection_t(W.T, e, b.reshape(1, _VOCAB))
    return out_t.T
